# Initial kernel scaffold; baseline (speedup 1.0000x reference)
import jax
import jax.numpy as jnp
from jax.experimental import pallas as pl


def kernel(data, pos, time, variables, batch, edge_index, dt, enc_W1, enc_b1, enc_W2, enc_b2, msg_W1, msg_b1, msg_W2, msg_b2, upd_W1, upd_b1, upd_W2, upd_b2, dec_W1, dec_b1, dec_W2, dec_b2):
    raise NotImplementedError("write your pallas kernel here")


# speedup vs baseline: 7.2307x; 7.2307x over previous
import functools

import jax
import jax.numpy as jnp
import numpy as np
from jax import lax
from jax.experimental import pallas as pl
from jax.experimental.pallas import tpu as pltpu
from jax.experimental.pallas import tpu_sc as plsc

N = 10000
E = 320000
TW = 25
NV = 3
D = 128
L = 6
NG = 16
T_MAX = 1.0

NC = 2
NS = 16
NW = NC * NS
EPW = E // NW
CH = 80
NCHUNK = EPW // CH
NPAD = 10240
ROWS_PER_TILE = NPAD // NS

_sc_mesh = plsc.VectorSubcoreMesh(core_axis_name="c", subcore_axis_name="s")


def _silu(x):
    return x / (1.0 + jnp.exp(-x))


@functools.partial(
    pl.kernel,
    mesh=_sc_mesh,
    out_type=[
        jax.ShapeDtypeStruct((E, D), jnp.float32),
        jax.ShapeDtypeStruct((E, D), jnp.float32),
    ],
    scratch_types=[
        pltpu.VMEM((NCHUNK, CH), jnp.int32),
        pltpu.VMEM((NCHUNK, CH), jnp.int32),
        pltpu.VMEM((CH, D), jnp.float32),
        pltpu.VMEM((CH, D), jnp.float32),
        pltpu.VMEM((CH, D), jnp.float32),
        pltpu.VMEM((CH, D), jnp.float32),
        pltpu.SemaphoreType.DMA,
        pltpu.SemaphoreType.DMA,
        pltpu.SemaphoreType.DMA,
    ],
)
def _sc_gather(A, B, dst3, src3, tA, tB,
               idxd, idxs, bufA0, bufA1, bufB0, bufB1, g0, g1, wsem):
    c = lax.axis_index("c")
    s = lax.axis_index("s")
    wid = s * NC + c
    base = wid * EPW
    pltpu.sync_copy(dst3.at[wid], idxd)
    pltpu.sync_copy(src3.at[wid], idxs)

    bufs = ((bufA0, bufB0, g0), (bufA1, bufB1, g1))

    def fire_gather(g, b):
        ba, bb, sem = bufs[b]
        pltpu.async_copy(A.at[idxd.at[g]], ba, sem)
        pltpu.async_copy(B.at[idxs.at[g]], bb, sem)

    def wait_gather(b):
        ba, bb, sem = bufs[b]
        pltpu.make_async_copy(A.at[pl.ds(0, CH)], ba, sem).wait()
        pltpu.make_async_copy(B.at[pl.ds(0, CH)], bb, sem).wait()

    def fire_write(g, b):
        ba, bb, _ = bufs[b]
        pltpu.async_copy(ba, tA.at[pl.ds(base + g * CH, CH)], wsem)
        pltpu.async_copy(bb, tB.at[pl.ds(base + g * CH, CH)], wsem)

    def wait_write():
        pltpu.make_async_copy(bufA0, tA.at[pl.ds(0, CH)], wsem).wait()
        pltpu.make_async_copy(bufB0, tB.at[pl.ds(0, CH)], wsem).wait()

    fire_gather(0, 0)

    def step(gg, carry):
        for b in range(2):
            g = gg * 2 + b
            if b == 0:
                @pl.when(gg >= 1)
                def _():
                    wait_write()
                fire_gather(g + 1, 1)
            else:
                wait_write()
                fire_gather(g + 1, 0)
            wait_gather(b)
            fire_write(g, b)
        return carry

    lax.fori_loop(0, NCHUNK // 2, step, None)
    wait_write()
    wait_gather(0)
    fire_write(NCHUNK - 1, 0)
    wait_write()


@functools.partial(
    pl.kernel,
    mesh=_sc_mesh,
    out_type=jax.ShapeDtypeStruct((NC, NPAD, D), jnp.float32),
    scratch_types=[
        pltpu.VMEM((NCHUNK, CH), jnp.int32),
        pltpu.VMEM((CH, D), jnp.float32),
        pltpu.VMEM((CH, D), jnp.float32),
        pltpu.VMEM_SHARED((NPAD, D), jnp.float32),
        pltpu.SemaphoreType.DMA,
        pltpu.SemaphoreType.DMA,
    ],
)
def _sc_scatter(m, dst3, zeros, out, idxd, rows0, rows1, acc, r0, r1):
    c = lax.axis_index("c")
    s = lax.axis_index("s")
    wid = s * NC + c
    base = wid * EPW
    pltpu.sync_copy(dst3.at[wid], idxd)
    row0 = s * ROWS_PER_TILE
    pltpu.sync_copy(zeros.at[pl.ds(row0, ROWS_PER_TILE)],
                    acc.at[pl.ds(row0, ROWS_PER_TILE)])
    plsc.subcore_barrier()

    bufs = ((rows0, r0), (rows1, r1))

    def fire_read(g, b):
        buf, sem = bufs[b]
        pltpu.async_copy(m.at[pl.ds(base + g * CH, CH)], buf, sem)

    def wait_read(b):
        buf, sem = bufs[b]
        pltpu.make_async_copy(m.at[pl.ds(0, CH)], buf, sem).wait()

    fire_read(0, 0)

    def step(gg, carry):
        for b in range(2):
            g = gg * 2 + b
            if b == 0:
                fire_read(g + 1, 1)
            else:
                fire_read(g + 1, 0)
            wait_read(b)
            buf, _ = bufs[b]
            pltpu.sync_copy(buf, acc.at[idxd.at[g]], add=True)
        return carry

    lax.fori_loop(0, NCHUNK // 2, step, None)
    wait_read(0)
    pltpu.sync_copy(rows0, acc.at[idxd.at[NCHUNK - 1]], add=True)
    plsc.subcore_barrier()
    pltpu.sync_copy(acc.at[pl.ds(row0, ROWS_PER_TILE)],
                    out.at[c, pl.ds(row0, ROWS_PER_TILE)])


@functools.partial(
    pl.kernel,
    mesh=_sc_mesh,
    out_type=jax.ShapeDtypeStruct((NC, NPAD, 16), jnp.float32),
    scratch_types=[
        pltpu.VMEM((NCHUNK, CH), jnp.int32),
        pltpu.VMEM((CH, 16), jnp.float32),
        pltpu.VMEM_SHARED((NPAD, 16), jnp.float32),
    ],
)
def _sc_degree(dst3, zeros16, ones16, out, idxd, onesv, acc):
    c = lax.axis_index("c")
    s = lax.axis_index("s")
    wid = s * NC + c
    pltpu.sync_copy(dst3.at[wid], idxd)
    pltpu.sync_copy(ones16, onesv)
    row0 = s * ROWS_PER_TILE
    pltpu.sync_copy(zeros16.at[pl.ds(row0, ROWS_PER_TILE)],
                    acc.at[pl.ds(row0, ROWS_PER_TILE)])
    plsc.subcore_barrier()

    def step(g, carry):
        pltpu.sync_copy(onesv, acc.at[idxd.at[g]], add=True)
        return carry

    lax.fori_loop(0, NCHUNK, step, None)
    plsc.subcore_barrier()
    pltpu.sync_copy(acc.at[pl.ds(row0, ROWS_PER_TILE)],
                    out.at[c, pl.ds(row0, ROWS_PER_TILE)])


BE = 4000


def _edge_body(tA_ref, tB_ref, W2_ref, b2_ref, m_ref):
    t = _silu(tA_ref[...] + tB_ref[...])
    mm = jnp.dot(t, W2_ref[...], preferred_element_type=jnp.float32) + b2_ref[...]
    m_ref[...] = _silu(mm)


def _edge_mlp(tA, tB, W2, b2):
    return pl.pallas_call(
        _edge_body,
        grid=(E // BE,),
        in_specs=[
            pl.BlockSpec((BE, D), lambda i: (i, 0)),
            pl.BlockSpec((BE, D), lambda i: (i, 0)),
            pl.BlockSpec((D, D), lambda i: (0, 0)),
            pl.BlockSpec((1, D), lambda i: (0, 0)),
        ],
        out_specs=pl.BlockSpec((BE, D), lambda i: (i, 0)),
        out_shape=jax.ShapeDtypeStruct((E, D), jnp.float32),
    )(tA, tB, W2, b2)


def _enc_body(data_ref, pos_ref, var_ref,
              We_d_ref, We_p_ref, We_v_ref, eb1_ref, eW2_ref, eb2_ref,
              h_ref, posn_ref):
    pos = pos_ref[...]
    posn = pos / jnp.max(pos)
    h1 = _silu(jnp.dot(data_ref[...], We_d_ref[...], preferred_element_type=jnp.float32)
               + jnp.dot(posn, We_p_ref[...], preferred_element_type=jnp.float32)
               + jnp.dot(var_ref[...], We_v_ref[...], preferred_element_type=jnp.float32)
               + eb1_ref[...])
    h_ref[...] = _silu(jnp.dot(h1, eW2_ref[...], preferred_element_type=jnp.float32)
                       + eb2_ref[...])
    posn_ref[...] = posn


def _encoder(data, pos, var, We_d, We_p, We_v, eb1, eW2, eb2):
    return pl.pallas_call(
        _enc_body,
        out_shape=[
            jax.ShapeDtypeStruct((N, D), jnp.float32),
            jax.ShapeDtypeStruct((N, 1), jnp.float32),
        ],
    )(data, pos, var, We_d, We_p, We_v, eb1, eW2, eb2)


def _norm_x(x, bc_ref, br_ref):
    iota_c = lax.broadcasted_iota(jnp.int32, (N, NG), 1)
    oh = (bc_ref[...] == iota_c).astype(jnp.float32)
    iota_r = lax.broadcasted_iota(jnp.int32, (NG, N), 0)
    ohT = (br_ref[...] == iota_r).astype(jnp.float32)
    ones = jnp.full((N, 1), 1.0, jnp.float32)
    cnt = jnp.dot(ohT, ones, preferred_element_type=jnp.float32)
    invc = 1.0 / jnp.maximum(cnt, 1.0)
    s1 = jnp.dot(ohT, x, preferred_element_type=jnp.float32)
    s2 = jnp.dot(ohT, x * x, preferred_element_type=jnp.float32)
    mean = s1 * invc
    vr = s2 * invc - mean * mean
    mb = jnp.dot(oh, mean, preferred_element_type=jnp.float32)
    vb = jnp.dot(oh, vr, preferred_element_type=jnp.float32)
    return (x - mb) * lax.rsqrt(vb + 1e-5)


def _update_h(h, p0, p1, d0, d1, var, bc_ref, br_ref,
              Wu_h, Wu_a, Wu_v, ub1, Wu2, ub2):
    deg = jnp.maximum(jnp.sum(d0 + d1, axis=1, keepdims=True), 1.0)
    agg = (p0 + p1) / deg
    u = _silu(jnp.dot(h, Wu_h, preferred_element_type=jnp.float32)
              + jnp.dot(agg, Wu_a, preferred_element_type=jnp.float32)
              + jnp.dot(var, Wu_v, preferred_element_type=jnp.float32)
              + ub1)
    u = _silu(jnp.dot(u, Wu2, preferred_element_type=jnp.float32) + ub2)
    return _norm_x(h + u, bc_ref, br_ref)


def _node_body(h_ref, p0_ref, p1_ref, d0_ref, d1_ref, var_ref, bc_ref, br_ref,
               Wu_h_ref, Wu_a_ref, Wu_v_ref, ub1_ref, Wu2_ref, ub2_ref,
               h_out_ref):
    h_out_ref[...] = _update_h(h_ref[...], p0_ref[...], p1_ref[...],
                               d0_ref[...], d1_ref[...], var_ref[...],
                               bc_ref, br_ref,
                               Wu_h_ref[...], Wu_a_ref[...], Wu_v_ref[...],
                               ub1_ref[...], Wu2_ref[...], ub2_ref[...])


def _node_update(h, p0, p1, d0, d1, var, bc, br, *weights):
    return pl.pallas_call(
        _node_body,
        out_shape=jax.ShapeDtypeStruct((N, D), jnp.float32),
    )(h, p0, p1, d0, d1, var, bc, br, *weights)


def _ab_body(h_ref, data_ref, posn_ref, var_ref,
             W1a_ref, W1b_ref, Wdat_ref, Wpos_ref, Wvar_ref, b1_ref,
             A_ref, B_ref):
    h = h_ref[...]
    pd = (jnp.dot(data_ref[...], Wdat_ref[...], preferred_element_type=jnp.float32)
          + jnp.dot(posn_ref[...], Wpos_ref[...], preferred_element_type=jnp.float32))
    A_ref[...] = (jnp.dot(h, W1a_ref[...], preferred_element_type=jnp.float32)
                  + pd
                  + jnp.dot(var_ref[...], Wvar_ref[...], preferred_element_type=jnp.float32)
                  + b1_ref[...])
    B_ref[...] = (jnp.dot(h, W1b_ref[...], preferred_element_type=jnp.float32)
                  - pd)


def _ab(h, data, posn, var, *weights):
    return pl.pallas_call(
        _ab_body,
        out_shape=[
            jax.ShapeDtypeStruct((N, D), jnp.float32),
            jax.ShapeDtypeStruct((N, D), jnp.float32),
        ],
    )(h, data, posn, var, *weights)


def _dec_body(h_ref, data_ref, dt_ref, K1_ref, b1r_ref, K2_ref, b2s_ref,
              out_ref):
    z1 = _silu(jnp.dot(h_ref[...], K1_ref[...], preferred_element_type=jnp.float32)
               + b1r_ref[...])
    diff = jnp.dot(z1, K2_ref[...], preferred_element_type=jnp.float32) + b2s_ref[...]
    steps = (lax.broadcasted_iota(jnp.int32, (1, TW), 1) + 1
             ).astype(jnp.float32) * dt_ref[...]
    out_ref[...] = data_ref[:, TW - 1:TW] + steps * diff


def _decoder(h, data, dt2, K1, b1r, K2, b2s):
    return pl.pallas_call(
        _dec_body,
        out_shape=jax.ShapeDtypeStruct((N, TW), jnp.float32),
    )(h, data, dt2, K1, b1r, K2, b2s)


_CONV1_LEN = (D - 16) // 3 + 1
_K1_COLS = 8 * _CONV1_LEN

_o1, _j1, _k1 = np.meshgrid(np.arange(8), np.arange(_CONV1_LEN), np.arange(16),
                            indexing="ij")
_K1_ROWS_IDX = (3 * _j1 + _k1).ravel()
_K1_COLS_IDX = (_o1 * _CONV1_LEN + _j1).ravel()
_K1_SRC = (_o1.ravel(), np.zeros_like(_o1).ravel(), _k1.ravel())

_o2, _t2, _k2 = np.meshgrid(np.arange(8), np.arange(TW), np.arange(14),
                            indexing="ij")
_K2_ROWS_IDX = (_o2 * _CONV1_LEN + _t2 + _k2).ravel()
_K2_COLS_IDX = _t2.ravel()
_K2_SRC = (np.zeros_like(_o2).ravel(), _o2.ravel(), _k2.ravel())


def kernel(data, pos, time, variables, batch, edge_index, dt,
           enc_W1, enc_b1, enc_W2, enc_b2,
           msg_W1, msg_b1, msg_W2, msg_b2,
           upd_W1, upd_b1, upd_W2, upd_b2,
           dec_W1, dec_b1, dec_W2, dec_b2):
    f32 = jnp.float32
    src = edge_index[0].astype(jnp.int32)
    dst = edge_index[1].astype(jnp.int32)
    dst3 = dst.reshape(NW, NCHUNK, CH)
    src3 = src.reshape(NW, NCHUNK, CH)
    var = jnp.concatenate((time / T_MAX, variables), axis=-1)
    bc = batch.astype(jnp.int32).reshape(N, 1)
    br = batch.astype(jnp.int32).reshape(1, N)

    zeros = jnp.zeros((NPAD, D), f32)
    zeros16 = jnp.zeros((NPAD, 16), f32)
    ones16 = jnp.ones((CH, 16), f32)

    We_d, We_p, We_v = enc_W1[:TW], enc_W1[TW:TW + 1], enc_W1[TW + 1:]
    eb1 = enc_b1.reshape(1, D)
    eb2 = enc_b2.reshape(1, D)

    def msg_splits(l):
        W1 = msg_W1[l]
        return (W1[:D], W1[D:2 * D], W1[2 * D:2 * D + TW],
                W1[2 * D + TW:2 * D + TW + 1], W1[2 * D + TW + 1:],
                msg_b1[l].reshape(1, D))

    def upd_splits(l):
        Wu = upd_W1[l]
        return (Wu[:D], Wu[D:2 * D], Wu[2 * D:], upd_b1[l].reshape(1, D),
                upd_W2[l], upd_b2[l].reshape(1, D))

    K1 = jnp.zeros((D, _K1_COLS), f32).at[_K1_ROWS_IDX, _K1_COLS_IDX].set(
        dec_W1[_K1_SRC])
    b1r = jnp.repeat(dec_b1, _CONV1_LEN).reshape(1, _K1_COLS)
    K2 = jnp.zeros((_K1_COLS, TW), f32).at[_K2_ROWS_IDX, _K2_COLS_IDX].add(
        dec_W2[_K2_SRC])
    b2s = jnp.broadcast_to(dec_b2.reshape(1, 1), (1, TW))
    dt2 = dt.reshape(1, 1)

    degp = _sc_degree(dst3, zeros16, ones16)
    d0, d1 = degp[0, :N], degp[1, :N]

    h, posn = _encoder(data, pos, var, We_d, We_p, We_v, eb1, enc_W2, eb2)
    A, B = _ab(h, data, posn, var, *msg_splits(0))

    for l in range(L):
        tA, tB = _sc_gather(A, B, dst3, src3)
        m = _edge_mlp(tA, tB, msg_W2[l], msg_b2[l].reshape(1, D))
        p = _sc_scatter(m, dst3, zeros)
        p0, p1 = p[0, :N], p[1, :N]
        h = _node_update(h, p0, p1, d0, d1, var, bc, br, *upd_splits(l))
        if l < L - 1:
            A, B = _ab(h, data, posn, var, *msg_splits(l + 1))
    return _decoder(h, data, dt2, K1, b1r, K2, b2s)
